# Initial kernel scaffold; baseline (speedup 1.0000x reference)
import jax
import jax.numpy as jnp
from jax.experimental import pallas as pl


def kernel(output_feats, input_map, superpoint, superpoint_center_xyz, edge_u_list, edge_v_list, W1, b1, W2, b2, We, be, Ws1, bs1, Ws2, bs2, Wo1, bo1, Wo2, bo2, Wc1, bc1, Wc2, bc2, Wz1, bz1, Wz2, bz2, Wp1, bp1, Wp2, bp2, Wq, Wk, Wv, Wf1, bf1, Wf2, bf2):
    raise NotImplementedError("write your pallas kernel here")


# speedup vs baseline: 1.2235x; 1.2235x over previous
import functools

import jax
import jax.numpy as jnp
from jax import lax
from jax.experimental import pallas as pl
from jax.experimental.pallas import tpu as pltpu
from jax.experimental.pallas import tpu_sc as plsc
import numpy as np

N_PTS = 200000
N_SP = 50000
E = 800000
D = 64


def _point_mlp_body(x_ref, w1_ref, b1_ref, w2_ref, b2_ref, o_ref):
    h = jnp.maximum(
        jnp.dot(x_ref[...], w1_ref[...], preferred_element_type=jnp.float32)
        + b1_ref[...],
        0.0,
    )
    o_ref[...] = (
        jnp.dot(h, w2_ref[...], preferred_element_type=jnp.float32) + b2_ref[...]
    )


def _tc_point_mlp(feats, W1, b1, W2, b2, bm=2048):
    n = feats.shape[0]
    c = W2.shape[1]
    return pl.pallas_call(
        _point_mlp_body,
        grid=(pl.cdiv(n, bm),),
        in_specs=[
            pl.BlockSpec((bm, D), lambda i: (i, 0)),
            pl.BlockSpec((D, D), lambda i: (0, 0)),
            pl.BlockSpec((1, D), lambda i: (0, 0)),
            pl.BlockSpec((D, c), lambda i: (0, 0)),
            pl.BlockSpec((1, c), lambda i: (0, 0)),
        ],
        out_specs=pl.BlockSpec((bm, c), lambda i: (i, 0)),
        out_shape=jax.ShapeDtypeStruct((n, c), jnp.float32),
    )(feats, W1, b1.reshape(1, -1), W2, b2.reshape(1, -1))


def _sp_dense_body(
    sums_ref, cnt_ref, we_ref, be_ref, wqkv_ref, wh1_ref, bh1_ref, wh2_ref,
    bh2_ref, ecc_ref, qkv_ref, heads_ref
):
    emb = sums_ref[...] / jnp.maximum(cnt_ref[...], 1.0)
    ecc = jnp.maximum(
        jnp.dot(emb, we_ref[...], preferred_element_type=jnp.float32) + be_ref[...],
        0.0,
    )
    ecc_ref[...] = ecc
    qkv_ref[...] = jnp.dot(ecc, wqkv_ref[...], preferred_element_type=jnp.float32)
    hid = jnp.maximum(
        jnp.dot(ecc, wh1_ref[...], preferred_element_type=jnp.float32) + bh1_ref[...],
        0.0,
    )
    heads_ref[...] = (
        jnp.dot(hid, wh2_ref[...], preferred_element_type=jnp.float32) + bh2_ref[...]
    )


def _tc_sp_dense(sums, counts, We, be, Wqkv, Wh1, bh1, Wh2, bh2, bm=1024):
    n = sums.shape[0]
    return pl.pallas_call(
        _sp_dense_body,
        grid=(pl.cdiv(n, bm),),
        in_specs=[
            pl.BlockSpec((bm, D), lambda i: (i, 0)),
            pl.BlockSpec((bm, 1), lambda i: (i, 0)),
            pl.BlockSpec((D, D), lambda i: (0, 0)),
            pl.BlockSpec((1, D), lambda i: (0, 0)),
            pl.BlockSpec((D, 192), lambda i: (0, 0)),
            pl.BlockSpec((D, 256), lambda i: (0, 0)),
            pl.BlockSpec((1, 256), lambda i: (0, 0)),
            pl.BlockSpec((256, 32), lambda i: (0, 0)),
            pl.BlockSpec((1, 32), lambda i: (0, 0)),
        ],
        out_specs=[
            pl.BlockSpec((bm, D), lambda i: (i, 0)),
            pl.BlockSpec((bm, 192), lambda i: (i, 0)),
            pl.BlockSpec((bm, 32), lambda i: (i, 0)),
        ],
        out_shape=[
            jax.ShapeDtypeStruct((n, D), jnp.float32),
            jax.ShapeDtypeStruct((n, 192), jnp.float32),
            jax.ShapeDtypeStruct((n, 32), jnp.float32),
        ],
    )(sums, counts.reshape(-1, 1), We, be.reshape(1, -1), Wqkv, Wh1,
      bh1.reshape(1, -1), Wh2, bh2.reshape(1, -1))


def _posenc_body(rel_ref, w1_ref, b1_ref, w2_ref, b2_ref, o_ref):
    h = jnp.maximum(
        jnp.dot(rel_ref[...], w1_ref[...], preferred_element_type=jnp.float32)
        + b1_ref[...],
        0.0,
    )
    o_ref[...] = (
        jnp.dot(h, w2_ref[...], preferred_element_type=jnp.float32) + b2_ref[...]
    )


def _tc_posenc(rel, Wp1p, bp1, Wp2, bp2, bm=8192):
    n = rel.shape[0]
    return pl.pallas_call(
        _posenc_body,
        grid=(pl.cdiv(n, bm),),
        in_specs=[
            pl.BlockSpec((bm, 8), lambda i: (i, 0)),
            pl.BlockSpec((8, 16), lambda i: (0, 0)),
            pl.BlockSpec((1, 16), lambda i: (0, 0)),
            pl.BlockSpec((16, 1), lambda i: (0, 0)),
            pl.BlockSpec((1, 1), lambda i: (0, 0)),
        ],
        out_specs=pl.BlockSpec((bm, 1), lambda i: (i, 0)),
        out_shape=jax.ShapeDtypeStruct((n, 1), jnp.float32),
    )(rel, Wp1p, bp1.reshape(1, -1), Wp2, bp2.reshape(1, -1))


def _total_body(tpart_ref, o_ref):
    t = tpart_ref[0, :] + tpart_ref[1, :]
    o_ref[...] = (1.0 / jnp.maximum(t, 1e-30)).reshape(1, -1)


def _tc_inv_total(tpart, bm=8192):
    n = tpart.shape[1]
    return pl.pallas_call(
        _total_body,
        grid=(pl.cdiv(n, bm),),
        in_specs=[pl.BlockSpec((2, bm), lambda i: (0, i))],
        out_specs=pl.BlockSpec((1, bm), lambda i: (0, i)),
        out_shape=jax.ShapeDtypeStruct((1, n), jnp.float32),
    )(tpart)


def _final_body(ecc_ref, res_ref, w1_ref, b1_ref, w2_ref, b2_ref, o_ref):
    sp_feat = ecc_ref[...] + res_ref[...]
    h = jnp.maximum(
        jnp.dot(sp_feat, w1_ref[...], preferred_element_type=jnp.float32)
        + b1_ref[...],
        0.0,
    )
    o_ref[...] = (
        jnp.dot(h, w2_ref[...], preferred_element_type=jnp.float32) + b2_ref[...]
    )


def _tc_final(ecc, res, Wf1, bf1, Wf2, bf2, bm=2048):
    n = ecc.shape[0]
    c = Wf2.shape[1]
    return pl.pallas_call(
        _final_body,
        grid=(pl.cdiv(n, bm),),
        in_specs=[
            pl.BlockSpec((bm, D), lambda i: (i, 0)),
            pl.BlockSpec((bm, D), lambda i: (i, 0)),
            pl.BlockSpec((D, D), lambda i: (0, 0)),
            pl.BlockSpec((1, D), lambda i: (0, 0)),
            pl.BlockSpec((D, c), lambda i: (0, 0)),
            pl.BlockSpec((1, c), lambda i: (0, 0)),
        ],
        out_specs=pl.BlockSpec((bm, c), lambda i: (i, 0)),
        out_shape=jax.ShapeDtypeStruct((n, c), jnp.float32),
    )(ecc, res, Wf1, bf1.reshape(1, -1), Wf2, bf2.reshape(1, -1))


def kernel(output_feats, input_map, superpoint, superpoint_center_xyz,
           edge_u_list, edge_v_list, W1, b1, W2, b2, We, be, Ws1, bs1, Ws2,
           bs2, Wo1, bo1, Wo2, bo2, Wc1, bc1, Wc2, bc2, Wz1, bz1, Wz2, bz2,
           Wp1, bp1, Wp2, bp2, Wq, Wk, Wv, Wf1, bf1, Wf2, bf2):
    f32 = jnp.float32

    feats = jnp.take(output_feats, input_map, axis=0)
    ones = jnp.ones((N_PTS,), dtype=f32)
    counts = jax.ops.segment_sum(ones, superpoint, num_segments=N_SP)
    sums = jax.ops.segment_sum(feats, superpoint, num_segments=N_SP)

    semantic_scores = _tc_point_mlp(feats, W1, b1, W2, b2)

    Wqkv = jnp.concatenate([Wq, Wk, Wv], axis=1)
    Wh1 = jnp.concatenate([Ws1, Wo1, Wc1, Wz1], axis=1)
    bh1 = jnp.concatenate([bs1, bo1, bc1, bz1], axis=0)
    Wh2 = jnp.zeros((256, 32), dtype=f32)
    Wh2 = Wh2.at[0:64, 0:20].set(Ws2)
    Wh2 = Wh2.at[64:128, 20:23].set(Wo2)
    Wh2 = Wh2.at[128:192, 23:24].set(Wc2)
    Wh2 = Wh2.at[192:256, 24:25].set(Wz2)
    bh2 = jnp.zeros((32,), dtype=f32)
    bh2 = bh2.at[0:20].set(bs2)
    bh2 = bh2.at[20:23].set(bo2)
    bh2 = bh2.at[23].set(bc2[0])
    bh2 = bh2.at[24].set(bz2[0])
    ecc, qkv, heads = _tc_sp_dense(sums, counts, We, be, Wqkv, Wh1, bh1, Wh2, bh2)
    q = qkv[:, 0:64]
    kk = qkv[:, 64:128]
    v = qkv[:, 128:192]
    sp_semantic_scores = heads[:, 0:20]
    pred_sp_offset_vectors = heads[:, 20:23]
    pred_sp_occupancy = heads[:, 23]
    pred_sp_ins_size = heads[:, 24]

    centers8 = jnp.pad(superpoint_center_xyz, ((0, 0), (0, 5)))
    rel = jnp.take(centers8, edge_u_list, axis=0) - jnp.take(
        centers8, edge_v_list, axis=0
    )

    Wp1p = jnp.pad(Wp1, ((0, 5), (0, 0)))
    pos_enc = _tc_posenc(rel, Wp1p, bp1, Wp2, bp2)[:, 0]

    affinity = (
        jnp.sum(
            jnp.take(q, edge_u_list, axis=0) * jnp.take(kk, edge_v_list, axis=0),
            axis=1,
        )
        / np.sqrt(D)
    ) * pos_enc
    ea = jnp.exp(affinity)
    tpart = jax.ops.segment_sum(ea, edge_u_list, num_segments=N_SP)
    tpart = jnp.stack([tpart, jnp.zeros_like(tpart)], axis=0)

    inv_total = _tc_inv_total(tpart)[0]

    edge_affinity = ea * jnp.take(inv_total, edge_u_list, axis=0)
    res = jax.ops.segment_sum(
        edge_affinity[:, None] * jnp.take(v, edge_v_list, axis=0),
        edge_u_list,
        num_segments=N_SP,
    )

    sp_discriminative_feats = _tc_final(ecc, res, Wf1, bf1, Wf2, bf2)

    return (semantic_scores, sp_semantic_scores, pred_sp_offset_vectors,
            pred_sp_occupancy, pred_sp_ins_size, edge_affinity,
            sp_discriminative_feats)


# speedup vs baseline: 1.2322x; 1.0071x over previous
import functools

import jax
import jax.numpy as jnp
from jax import lax
from jax.experimental import pallas as pl
from jax.experimental.pallas import tpu as pltpu
from jax.experimental.pallas import tpu_sc as plsc
import numpy as np

N_PTS = 200000
N_SP = 50000
E = 800000
D = 64


def _point_mlp_body(x_ref, w1_ref, b1_ref, w2_ref, b2_ref, o_ref):
    h = jnp.maximum(
        jnp.dot(x_ref[...], w1_ref[...], preferred_element_type=jnp.float32)
        + b1_ref[...],
        0.0,
    )
    o_ref[...] = (
        jnp.dot(h, w2_ref[...], preferred_element_type=jnp.float32) + b2_ref[...]
    )


def _tc_point_mlp(feats, W1, b1, W2, b2, bm=2048):
    n = feats.shape[0]
    c = W2.shape[1]
    return pl.pallas_call(
        _point_mlp_body,
        grid=(pl.cdiv(n, bm),),
        in_specs=[
            pl.BlockSpec((bm, D), lambda i: (i, 0)),
            pl.BlockSpec((D, D), lambda i: (0, 0)),
            pl.BlockSpec((1, D), lambda i: (0, 0)),
            pl.BlockSpec((D, c), lambda i: (0, 0)),
            pl.BlockSpec((1, c), lambda i: (0, 0)),
        ],
        out_specs=pl.BlockSpec((bm, c), lambda i: (i, 0)),
        out_shape=jax.ShapeDtypeStruct((n, c), jnp.float32),
    )(feats, W1, b1.reshape(1, -1), W2, b2.reshape(1, -1))


def _sp_dense_body(
    sums0_ref, sums1_ref, cnt_ref, we_ref, be_ref, wqkv_ref, wh1_ref, bh1_ref,
    wh2_ref, bh2_ref, ecc_ref, qkv_ref, heads_ref
):
    cnt = jnp.sum(cnt_ref[...], axis=1, keepdims=True)
    emb = (sums0_ref[...] + sums1_ref[...]) / jnp.maximum(cnt, 1.0)
    ecc = jnp.maximum(
        jnp.dot(emb, we_ref[...], preferred_element_type=jnp.float32) + be_ref[...],
        0.0,
    )
    ecc_ref[...] = ecc
    qkv_ref[...] = jnp.dot(ecc, wqkv_ref[...], preferred_element_type=jnp.float32)
    hid = jnp.maximum(
        jnp.dot(ecc, wh1_ref[...], preferred_element_type=jnp.float32) + bh1_ref[...],
        0.0,
    )
    heads_ref[...] = (
        jnp.dot(hid, wh2_ref[...], preferred_element_type=jnp.float32) + bh2_ref[...]
    )


def _tc_sp_dense(sums0, sums1, counts, We, be, Wqkv, Wh1, bh1, Wh2, bh2,
                 bm=1024):
    n = sums0.shape[0]
    return pl.pallas_call(
        _sp_dense_body,
        grid=(pl.cdiv(n, bm),),
        in_specs=[
            pl.BlockSpec((bm, D), lambda i: (i, 0)),
            pl.BlockSpec((bm, D), lambda i: (i, 0)),
            pl.BlockSpec((bm, 2), lambda i: (i, 0)),
            pl.BlockSpec((D, D), lambda i: (0, 0)),
            pl.BlockSpec((1, D), lambda i: (0, 0)),
            pl.BlockSpec((D, 192), lambda i: (0, 0)),
            pl.BlockSpec((D, 256), lambda i: (0, 0)),
            pl.BlockSpec((1, 256), lambda i: (0, 0)),
            pl.BlockSpec((256, 32), lambda i: (0, 0)),
            pl.BlockSpec((1, 32), lambda i: (0, 0)),
        ],
        out_specs=[
            pl.BlockSpec((bm, D), lambda i: (i, 0)),
            pl.BlockSpec((bm, 192), lambda i: (i, 0)),
            pl.BlockSpec((bm, 32), lambda i: (i, 0)),
        ],
        out_shape=[
            jax.ShapeDtypeStruct((n, D), jnp.float32),
            jax.ShapeDtypeStruct((n, 192), jnp.float32),
            jax.ShapeDtypeStruct((n, 32), jnp.float32),
        ],
    )(sums0, sums1, counts, We, be.reshape(1, -1), Wqkv, Wh1,
      bh1.reshape(1, -1), Wh2, bh2.reshape(1, -1))


def _edge_ea_body(qg_ref, kg_ref, cu_ref, cv_ref, w1_ref, b1_ref, w2_ref,
                  b2_ref, o_ref):
    rel = cu_ref[...] - cv_ref[...]
    h = jnp.maximum(
        jnp.dot(rel, w1_ref[...], preferred_element_type=jnp.float32)
        + b1_ref[...],
        0.0,
    )
    pe = jnp.dot(h, w2_ref[...], preferred_element_type=jnp.float32) + b2_ref[...]
    aff = jnp.sum(qg_ref[...] * kg_ref[...], axis=1, keepdims=True)
    ea = jnp.exp(aff * (1.0 / np.sqrt(D)) * pe)
    o_ref[...] = jnp.broadcast_to(ea, (ea.shape[0], 16))


def _tc_edge_ea(qg, kg, cu, cv, Wp1p, bp1, Wp2, bp2, bm=2048):
    n = qg.shape[0]
    return pl.pallas_call(
        _edge_ea_body,
        grid=(pl.cdiv(n, bm),),
        in_specs=[
            pl.BlockSpec((bm, D), lambda i: (i, 0)),
            pl.BlockSpec((bm, D), lambda i: (i, 0)),
            pl.BlockSpec((bm, 8), lambda i: (i, 0)),
            pl.BlockSpec((bm, 8), lambda i: (i, 0)),
            pl.BlockSpec((8, 16), lambda i: (0, 0)),
            pl.BlockSpec((1, 16), lambda i: (0, 0)),
            pl.BlockSpec((16, 1), lambda i: (0, 0)),
            pl.BlockSpec((1, 1), lambda i: (0, 0)),
        ],
        out_specs=pl.BlockSpec((bm, 16), lambda i: (i, 0)),
        out_shape=jax.ShapeDtypeStruct((n, 16), jnp.float32),
    )(qg, kg, cu, cv, Wp1p, bp1.reshape(1, -1), Wp2, bp2.reshape(1, -1))


def _inv8_body(tpart_ref, o_ref):
    t = tpart_ref[0, :, 0] + tpart_ref[1, :, 0]
    inv = 1.0 / jnp.maximum(t, 1e-30)
    o_ref[...] = jnp.broadcast_to(inv[:, None], (inv.shape[0], 8))


def _tc_inv8(tpart, bm=8192):
    n = tpart.shape[1]
    return pl.pallas_call(
        _inv8_body,
        grid=(pl.cdiv(n, bm),),
        in_specs=[pl.BlockSpec((2, bm, 16), lambda i: (0, i, 0))],
        out_specs=pl.BlockSpec((bm, 8), lambda i: (i, 0)),
        out_shape=jax.ShapeDtypeStruct((n, 8), jnp.float32),
    )(tpart)


def _edge_out_body(ea_ref, vg_ref, ivg_ref, contrib_ref, eaff_ref):
    w = ea_ref[:, 0:1] * ivg_ref[:, 0:1]
    eaff_ref[...] = w
    contrib_ref[...] = w * vg_ref[...]


def _tc_edge_out(ea4, vg, ivg, bm=2048):
    n = ea4.shape[0]
    return pl.pallas_call(
        _edge_out_body,
        grid=(pl.cdiv(n, bm),),
        in_specs=[
            pl.BlockSpec((bm, 16), lambda i: (i, 0)),
            pl.BlockSpec((bm, D), lambda i: (i, 0)),
            pl.BlockSpec((bm, 8), lambda i: (i, 0)),
        ],
        out_specs=[
            pl.BlockSpec((bm, D), lambda i: (i, 0)),
            pl.BlockSpec((bm, 1), lambda i: (i, 0)),
        ],
        out_shape=[
            jax.ShapeDtypeStruct((n, D), jnp.float32),
            jax.ShapeDtypeStruct((n, 1), jnp.float32),
        ],
    )(ea4, vg, ivg)


def _final_body(ecc_ref, res0_ref, res1_ref, w1_ref, b1_ref, w2_ref, b2_ref,
                o_ref):
    sp_feat = ecc_ref[...] + res0_ref[...] + res1_ref[...]
    h = jnp.maximum(
        jnp.dot(sp_feat, w1_ref[...], preferred_element_type=jnp.float32)
        + b1_ref[...],
        0.0,
    )
    o_ref[...] = (
        jnp.dot(h, w2_ref[...], preferred_element_type=jnp.float32) + b2_ref[...]
    )


def _tc_final(ecc, res0, res1, Wf1, bf1, Wf2, bf2, bm=2048):
    n = ecc.shape[0]
    c = Wf2.shape[1]
    return pl.pallas_call(
        _final_body,
        grid=(pl.cdiv(n, bm),),
        in_specs=[
            pl.BlockSpec((bm, D), lambda i: (i, 0)),
            pl.BlockSpec((bm, D), lambda i: (i, 0)),
            pl.BlockSpec((bm, D), lambda i: (i, 0)),
            pl.BlockSpec((D, D), lambda i: (0, 0)),
            pl.BlockSpec((1, D), lambda i: (0, 0)),
            pl.BlockSpec((D, c), lambda i: (0, 0)),
            pl.BlockSpec((1, c), lambda i: (0, 0)),
        ],
        out_specs=pl.BlockSpec((bm, c), lambda i: (i, 0)),
        out_shape=jax.ShapeDtypeStruct((n, c), jnp.float32),
    )(ecc, res0, res1, Wf1, bf1.reshape(1, -1), Wf2, bf2.reshape(1, -1))


NC, NS = 2, 16
NW = NC * NS
CH = 128

_MESH = plsc.VectorSubcoreMesh(
    core_axis_name="c", subcore_axis_name="s", num_cores=NC, num_subcores=NS
)
_SC_PARAMS = pltpu.CompilerParams(use_tc_tiling_on_sc=False, needs_layout_passes=False)


def _sc_gather_n(idx, *tables):
    n = idx.shape[0]
    nfull = n // CH
    tail = n - nfull * CH
    outs = [jax.ShapeDtypeStruct((n, t.shape[1]), t.dtype) for t in tables]
    scratch = [pltpu.VMEM((CH,), jnp.int32)]
    scratch += [pltpu.VMEM((CH, t.shape[1]), t.dtype) for t in tables]
    if tail:
        scratch += [pltpu.VMEM((tail,), jnp.int32)]
        scratch += [pltpu.VMEM((tail, t.shape[1]), t.dtype) for t in tables]
    scratch += [pltpu.SemaphoreType.DMA]
    nt = len(tables)

    @functools.partial(
        pl.kernel, mesh=_MESH, out_type=outs, scratch_types=scratch,
        compiler_params=_SC_PARAMS,
    )
    def k(*refs):
        idx_hbm = refs[0]
        tab = refs[1 : 1 + nt]
        out = refs[1 + nt : 1 + 2 * nt]
        idx_v = refs[1 + 2 * nt]
        bufs = refs[2 + 2 * nt : 2 + 3 * nt]
        if tail:
            tidx_v = refs[2 + 3 * nt]
            tbufs = refs[3 + 3 * nt : 3 + 4 * nt]
            sem = refs[3 + 4 * nt]
        else:
            sem = refs[2 + 3 * nt]
        wid = lax.axis_index("s") * NC + lax.axis_index("c")
        trips = (nfull - wid + NW - 1) // NW

        def body(i, carry):
            base = (wid + i * NW) * CH
            pltpu.sync_copy(idx_hbm.at[pl.ds(base, CH)], idx_v)
            for t in range(nt):
                pltpu.async_copy(tab[t].at[idx_v], bufs[t], sem).wait()
                pltpu.sync_copy(bufs[t], out[t].at[pl.ds(base, CH)])
            return carry

        lax.fori_loop(0, trips, body, 0)

        if tail:
            @pl.when(wid == NW - 1)
            def _():
                base = nfull * CH
                pltpu.sync_copy(idx_hbm.at[pl.ds(base, tail)], tidx_v)
                for t in range(nt):
                    pltpu.async_copy(tab[t].at[tidx_v], tbufs[t], sem).wait()
                    pltpu.sync_copy(tbufs[t], out[t].at[pl.ds(base, tail)])

    return k(idx, *tables)


def _dedup_scan(idxb, buf, sidx, ch, W, n_seg, also=None):
    iota = lax.iota(jnp.int32, 16)

    step = 1
    while step < ch:
        for j0 in range(0, ch, 16):
            ids_j = idxb[pl.ds(8 + j0, 16)]
            ids_s = plsc.load_gather(idxb, [iota + (8 + j0) + step])
            mask = ids_j == ids_s
            anyg = jnp.any(mask)

            @pl.when(anyg)
            def _(j0=j0, mask=mask, step=step):
                rows_a = iota + j0
                rows_b = rows_a + step

                def colbody(cc, carry):
                    colv = jnp.zeros((16,), jnp.int32) + cc
                    for b in ((buf,) if also is None else (buf, also)):
                        va = plsc.load_gather(b, [rows_a, colv])
                        vb = plsc.load_gather(b, [rows_b, colv])
                        plsc.store_scatter(
                            b, [rows_a, colv], va + jnp.where(mask, vb, 0.0)
                        )
                    return carry

                lax.fori_loop(0, W, colbody, 0)
        step *= 2
    for j0 in range(0, ch, 16):
        ids_j = idxb[pl.ds(8 + j0, 16)]
        prev = plsc.load_gather(idxb, [iota + (7 + j0)])
        sidx[pl.ds(j0, 16)] = jnp.where(ids_j != prev, ids_j, n_seg + j0 + iota)


def _fill_sentinel(idxb):
    for t in range(idxb.shape[0] // 16):
        idxb[pl.ds(16 * t, 16)] = jnp.full((16,), -1, jnp.int32)


def _sc_scatter_rows(rows4, idx, n_seg, zeros):
    n = idx.shape[0]
    nfull = n // CH
    tail = n - nfull * CH
    stripe = n_seg // NS
    assert n_seg % NS == 0 and zeros.shape[0] >= stripe
    scratch = [
        pltpu.VMEM((208,), jnp.int32),
        pltpu.VMEM((CH + 64, 16), jnp.float32),
        pltpu.VMEM((CH + 64, 16), jnp.float32),
        pltpu.VMEM((CH,), jnp.int32),
        pltpu.VMEM_SHARED((n_seg + CH, 16), jnp.float32),
        pltpu.VMEM_SHARED((n_seg + CH, 16), jnp.float32),
    ]
    if tail:
        scratch += [
            pltpu.VMEM((208,), jnp.int32),
            pltpu.VMEM((tail + 64, 16), jnp.float32),
            pltpu.VMEM((tail + 64, 16), jnp.float32),
            pltpu.VMEM((tail,), jnp.int32),
        ]

    @functools.partial(
        pl.kernel, mesh=_MESH,
        out_type=jax.ShapeDtypeStruct((2 * NC, n_seg, 16), jnp.float32),
        scratch_types=scratch, compiler_params=_SC_PARAMS,
    )
    def k(*refs):
        (rows_hbm, idx_hbm, zeros_hbm, out_sums, idxb, buf0, buf1, sidx,
         acc0, acc1) = refs[:10]
        bufs = (buf0, buf1)
        accs = (acc0, acc1)
        if tail:
            tidxb, tbuf0, tbuf1, tsidx = refs[10:14]
            tbufs = (tbuf0, tbuf1)
        s = lax.axis_index("s")
        c = lax.axis_index("c")
        wid = s * NC + c
        r0 = s * stripe

        for q in range(2):
            pltpu.sync_copy(zeros_hbm.at[pl.ds(0, stripe), pl.ds(0, 16)],
                            accs[q].at[pl.ds(r0, stripe)])
            pltpu.sync_copy(zeros_hbm.at[pl.ds(0, 64), pl.ds(0, 16)],
                            bufs[q].at[pl.ds(CH, 64)])
            if tail:
                pltpu.sync_copy(zeros_hbm.at[pl.ds(0, 64), pl.ds(0, 16)],
                                tbufs[q].at[pl.ds(tail, 64)])
        _fill_sentinel(idxb)
        if tail:
            _fill_sentinel(tidxb)
        plsc.subcore_barrier()

        def body(i, carry):
            base = (wid + i * NW) * CH
            pltpu.sync_copy(idx_hbm.at[pl.ds(base, CH)], idxb.at[pl.ds(8, CH)])
            for q in range(2):
                pltpu.sync_copy(
                    rows_hbm.at[pl.ds((2 * c + q) * n + base, CH)],
                    bufs[q].at[pl.ds(0, CH)],
                )
            _dedup_scan(idxb, bufs[0], sidx, CH, 16, n_seg, also=bufs[1])
            for q in range(2):
                pltpu.sync_copy(bufs[q].at[pl.ds(0, CH)], accs[q].at[sidx],
                                add=True)
            return carry

        trips = (nfull - wid + NW - 1) // NW
        lax.fori_loop(0, trips, body, 0)

        if tail:
            @pl.when(s == NS - 1)
            def _():
                base = nfull * CH
                pltpu.sync_copy(idx_hbm.at[pl.ds(base, tail)],
                                tidxb.at[pl.ds(8, tail)])
                for q in range(2):
                    pltpu.sync_copy(
                        rows_hbm.at[pl.ds((2 * c + q) * n + base, tail)],
                        tbufs[q].at[pl.ds(0, tail)],
                    )
                _dedup_scan(tidxb, tbufs[0], tsidx, tail, 16, n_seg,
                            also=tbufs[1])
                for q in range(2):
                    pltpu.sync_copy(tbufs[q].at[pl.ds(0, tail)],
                                    accs[q].at[tsidx], add=True)

        plsc.subcore_barrier()
        for q in range(2):
            pltpu.sync_copy(accs[q].at[pl.ds(r0, stripe)],
                            out_sums.at[2 * c + q, pl.ds(r0, stripe)])

    return k(rows4, idx, zeros)


def _split_cols(rows):
    n = rows.shape[0]
    return jnp.transpose(rows.reshape(n, 4, 16), (1, 0, 2)).reshape(4 * n, 16)


def _unsplit_cols(out4):
    s = out4.shape[1]
    return jnp.transpose(out4, (1, 0, 2)).reshape(s, 64)


def _sc_scatter1(idx, n_seg, zeros16, ones16, vals16=None):
    n = idx.shape[0]
    nfull = n // CH
    tail = n - nfull * CH
    stripe = n_seg // NS
    assert n_seg % NS == 0
    use_ones = vals16 is None
    scratch = [
        pltpu.VMEM((208,), jnp.int32),
        pltpu.VMEM((CH + 64, 16), jnp.float32),
        pltpu.VMEM((CH,), jnp.int32),
        pltpu.VMEM_SHARED((n_seg + CH, 16), jnp.float32),
    ]
    if tail:
        scratch += [
            pltpu.VMEM((208,), jnp.int32),
            pltpu.VMEM((tail + 64, 16), jnp.float32),
            pltpu.VMEM((tail,), jnp.int32),
        ]

    @functools.partial(
        pl.kernel, mesh=_MESH,
        out_type=jax.ShapeDtypeStruct((NC, n_seg, 16), jnp.float32),
        scratch_types=scratch, compiler_params=_SC_PARAMS,
    )
    def k(*refs):
        if use_ones:
            (idx_hbm, zeros_hbm, ones_hbm, out) = refs[:4]
            rest = refs[4:]
        else:
            (vals_hbm, idx_hbm, zeros_hbm, ones_hbm, out) = refs[:5]
            rest = refs[5:]
        (idxb, buf, sidx, acc) = rest[:4]
        if tail:
            tidxb, tbuf, tsidx = rest[4:7]
        s = lax.axis_index("s")
        c = lax.axis_index("c")
        wid = s * NC + c
        r0 = s * stripe

        pltpu.sync_copy(zeros_hbm, acc.at[pl.ds(r0, stripe)])
        _fill_sentinel(idxb)
        pltpu.sync_copy(zeros_hbm.at[pl.ds(0, 64)], buf.at[pl.ds(CH, 64)])
        if tail:
            _fill_sentinel(tidxb)
            pltpu.sync_copy(zeros_hbm.at[pl.ds(0, 64)], tbuf.at[pl.ds(tail, 64)])
        plsc.subcore_barrier()

        def body(i, carry):
            base = (wid + i * NW) * CH
            pltpu.sync_copy(idx_hbm.at[pl.ds(base, CH)], idxb.at[pl.ds(8, CH)])
            if use_ones:
                pltpu.sync_copy(ones_hbm, buf.at[pl.ds(0, CH)])
            else:
                pltpu.sync_copy(vals_hbm.at[pl.ds(base, CH)], buf.at[pl.ds(0, CH)])
            _dedup_scan(idxb, buf, sidx, CH, 16, n_seg)
            pltpu.sync_copy(buf.at[pl.ds(0, CH)], acc.at[sidx], add=True)
            return carry

        trips = (nfull - wid + NW - 1) // NW
        lax.fori_loop(0, trips, body, 0)

        if tail:
            @pl.when(wid == NW - 1)
            def _():
                base = nfull * CH
                pltpu.sync_copy(idx_hbm.at[pl.ds(base, tail)],
                                tidxb.at[pl.ds(8, tail)])
                if use_ones:
                    pltpu.sync_copy(ones_hbm.at[pl.ds(0, tail)],
                                    tbuf.at[pl.ds(0, tail)])
                else:
                    pltpu.sync_copy(vals_hbm.at[pl.ds(base, tail)],
                                    tbuf.at[pl.ds(0, tail)])
                _dedup_scan(tidxb, tbuf, tsidx, tail, 16, n_seg)
                pltpu.sync_copy(tbuf.at[pl.ds(0, tail)], acc.at[tsidx], add=True)

        plsc.subcore_barrier()
        pltpu.sync_copy(acc.at[pl.ds(r0, stripe)], out.at[c, pl.ds(r0, stripe)])

    if use_ones:
        return k(idx, zeros16, ones16)
    return k(vals16, idx, zeros16, ones16)


def kernel(output_feats, input_map, superpoint, superpoint_center_xyz,
           edge_u_list, edge_v_list, W1, b1, W2, b2, We, be, Ws1, bs1, Ws2,
           bs2, Wo1, bo1, Wo2, bo2, Wc1, bc1, Wc2, bc2, Wz1, bz1, Wz2, bz2,
           Wp1, bp1, Wp2, bp2, Wq, Wk, Wv, Wf1, bf1, Wf2, bf2):
    f32 = jnp.float32
    zeros = jnp.zeros((N_SP // NS, 32), dtype=f32)
    zeros16 = jnp.zeros((N_SP // NS, 16), dtype=f32)
    ones16 = jnp.ones((CH, 16), dtype=f32)

    (feats,) = _sc_gather_n(input_map, output_feats)
    sums_pc = jnp.concatenate(
        [_sc_scatter1(superpoint, N_SP, zeros16, ones16,
                      feats[:, 16 * q:16 * q + 16]) for q in range(4)],
        axis=2,
    )
    cpart = _sc_scatter1(superpoint, N_SP, zeros16, ones16)
    counts2 = jnp.transpose(cpart[:, :, 0])

    semantic_scores = _tc_point_mlp(feats, W1, b1, W2, b2)

    Wqkv = jnp.concatenate([Wq, Wk, Wv], axis=1)
    Wh1 = jnp.concatenate([Ws1, Wo1, Wc1, Wz1], axis=1)
    bh1 = jnp.concatenate([bs1, bo1, bc1, bz1], axis=0)
    Wh2 = jnp.zeros((256, 32), dtype=f32)
    Wh2 = Wh2.at[0:64, 0:20].set(Ws2)
    Wh2 = Wh2.at[64:128, 20:23].set(Wo2)
    Wh2 = Wh2.at[128:192, 23:24].set(Wc2)
    Wh2 = Wh2.at[192:256, 24:25].set(Wz2)
    bh2 = jnp.zeros((32,), dtype=f32)
    bh2 = bh2.at[0:20].set(bs2)
    bh2 = bh2.at[20:23].set(bo2)
    bh2 = bh2.at[23].set(bc2[0])
    bh2 = bh2.at[24].set(bz2[0])
    ecc, qkv, heads = _tc_sp_dense(sums_pc[0], sums_pc[1], counts2, We, be,
                                   Wqkv, Wh1, bh1, Wh2, bh2)
    q = qkv[:, 0:64]
    kk = qkv[:, 64:128]
    v = qkv[:, 128:192]
    sp_semantic_scores = heads[:, 0:20]
    pred_sp_offset_vectors = heads[:, 20:23]
    pred_sp_occupancy = heads[:, 23]
    pred_sp_ins_size = heads[:, 24]

    centers8 = jnp.pad(superpoint_center_xyz, ((0, 0), (0, 5)))
    qg, cu = _sc_gather_n(edge_u_list, q, centers8)
    kg, vg, cv = _sc_gather_n(edge_v_list, kk, v, centers8)

    Wp1p = jnp.pad(Wp1, ((0, 5), (0, 0)))
    ea4 = _tc_edge_ea(qg, kg, cu, cv, Wp1p, bp1, Wp2, bp2)

    tpart = _sc_scatter1(edge_u_list, N_SP, zeros16, ones16, ea4)

    inv8 = _tc_inv8(tpart)

    (ivg,) = _sc_gather_n(edge_u_list, inv8)
    contrib, eaff = _tc_edge_out(ea4, vg, ivg)
    edge_affinity = eaff[:, 0]
    res_pc = jnp.concatenate(
        [_sc_scatter1(edge_u_list, N_SP, zeros16, ones16,
                      contrib[:, 16 * q:16 * q + 16]) for q in range(4)],
        axis=2,
    )

    sp_discriminative_feats = _tc_final(ecc, res_pc[0], res_pc[1], Wf1,
                                        bf1, Wf2, bf2)

    return (semantic_scores, sp_semantic_scores, pred_sp_offset_vectors,
            pred_sp_occupancy, pred_sp_ins_size, edge_affinity,
            sp_discriminative_feats)
